# Initial kernel scaffold; baseline (speedup 1.0000x reference)
import jax
import jax.numpy as jnp
from jax.experimental import pallas as pl


def kernel(x, pe_weight):
    raise NotImplementedError("write your pallas kernel here")


# speedup vs baseline: 1.6970x; 1.6970x over previous
import jax
import jax.numpy as jnp
from jax.experimental import pallas as pl

_TL = 512


def _pe_add_kernel(x_ref, pe_ref, o_ref):
    o_ref[...] = x_ref[...] + pe_ref[...]


def kernel(x, pe_weight):
    b, l, d = x.shape
    grid = (l // _TL, b)
    return pl.pallas_call(
        _pe_add_kernel,
        grid=grid,
        in_specs=[
            pl.BlockSpec((1, _TL, d), lambda i, j: (j, i, 0)),
            pl.BlockSpec((_TL, d), lambda i, j: (i, 0)),
        ],
        out_specs=pl.BlockSpec((1, _TL, d), lambda i, j: (j, i, 0)),
        out_shape=jax.ShapeDtypeStruct((b, l, d), x.dtype),
    )(x, pe_weight)


# speedup vs baseline: 1.8809x; 1.1084x over previous
import jax
import jax.numpy as jnp
from jax.experimental import pallas as pl

_TL = 1024


def _pe_add_kernel(x_ref, pe_ref, o_ref):
    o_ref[...] = x_ref[...] + pe_ref[...]


def kernel(x, pe_weight):
    b, l, d = x.shape
    grid = (l // _TL, b)
    return pl.pallas_call(
        _pe_add_kernel,
        grid=grid,
        in_specs=[
            pl.BlockSpec((1, _TL, d), lambda i, j: (j, i, 0)),
            pl.BlockSpec((_TL, d), lambda i, j: (i, 0)),
        ],
        out_specs=pl.BlockSpec((1, _TL, d), lambda i, j: (j, i, 0)),
        out_shape=jax.ShapeDtypeStruct((b, l, d), x.dtype),
    )(x, pe_weight)


# speedup vs baseline: 1.9860x; 1.0559x over previous
import jax
import jax.numpy as jnp
from jax.experimental import pallas as pl

_TL = 2048


def _pe_add_kernel(x_ref, pe_ref, o_ref):
    o_ref[...] = x_ref[...] + pe_ref[...]


def kernel(x, pe_weight):
    b, l, d = x.shape
    grid = (l // _TL, b)
    return pl.pallas_call(
        _pe_add_kernel,
        grid=grid,
        in_specs=[
            pl.BlockSpec((1, _TL, d), lambda i, j: (j, i, 0)),
            pl.BlockSpec((_TL, d), lambda i, j: (i, 0)),
        ],
        out_specs=pl.BlockSpec((1, _TL, d), lambda i, j: (j, i, 0)),
        out_shape=jax.ShapeDtypeStruct((b, l, d), x.dtype),
    )(x, pe_weight)
